# Initial kernel scaffold; baseline (speedup 1.0000x reference)
import jax
import jax.numpy as jnp
from jax.experimental import pallas as pl


def kernel(h, edge_index, W, A, Wrow, brow, gamma_row, beta_row, Wcol, bcol, gamma_col, beta_col):
    raise NotImplementedError("write your pallas kernel here")


# speedup vs baseline: 10.3439x; 10.3439x over previous
import functools

import jax
import jax.numpy as jnp
from jax import lax
from jax.experimental import pallas as pl
from jax.experimental.pallas import tpu as pltpu
from jax.experimental.pallas import tpu_sc as plsc

N = 10000
K = 32
D = 128
E = N * K
NW = 32
GN = 16
NG = N // GN
EPG = GN * K


def _dense_body(h_ref, wt_ref, baux_ref, z_ref, aux_ref):
    z = jnp.dot(h_ref[...], wt_ref[...], preferred_element_type=jnp.float32)
    z_ref[...] = z
    aux_ref[0] = lax.dot_general(
        baux_ref[...], z, (((1,), (1,)), ((), ())),
        preferred_element_type=jnp.float32)


def _dense(h, wt, baux):
    nt = 10
    tb = N // nt
    return pl.pallas_call(
        _dense_body,
        grid=(nt,),
        in_specs=[
            pl.BlockSpec((tb, D), lambda i: (i, 0)),
            pl.BlockSpec((D, D), lambda i: (0, 0)),
            pl.BlockSpec((8, D), lambda i: (0, 0)),
        ],
        out_specs=[
            pl.BlockSpec((tb, D), lambda i: (i, 0)),
            pl.BlockSpec((1, 8, tb), lambda i: (i, 0, 0)),
        ],
        out_shape=[
            jax.ShapeDtypeStruct((N, D), jnp.float32),
            jax.ShapeDtypeStruct((nt, 8, tb), jnp.float32),
        ],
    )(h, wt, baux)


def _sc_body(src_hbm, aux_hbm, z_hbm, wcol_hbm, row_hbm, col_hbm,
             s_st, t_st, u0_st, u1_st, wcol_st, src_st,
             e_st, u0g_st, u1g_st, w_st, row_st, rows_v, col_st, sem):
    wid = lax.axis_index("s") * 2 + lax.axis_index("c")

    pltpu.sync_copy(aux_hbm.at[0], s_st)
    pltpu.sync_copy(aux_hbm.at[1], t_st)
    pltpu.sync_copy(aux_hbm.at[2], u0_st)
    pltpu.sync_copy(aux_hbm.at[3], u1_st)
    pltpu.sync_copy(wcol_hbm, wcol_st)

    iota = lax.iota(jnp.int32, 16)
    iota32 = iota * K

    ngroups = jnp.where(wid < NG - (NG // NW) * NW, NG // NW + 1, NG // NW)

    def group_body(gi, carry):
        g = wid + gi * NW
        nbase = g * GN
        ebase = nbase * K

        pltpu.sync_copy(src_hbm.at[pl.ds(ebase, EPG)], src_st)

        copies = []
        for b in range(4):
            cp = pltpu.make_async_copy(
                z_hbm.at[src_st.at[pl.ds(b * 128, 128)]],
                rows_v.at[pl.ds(b * 128, 128)],
                sem)
            cp.start()
            copies.append(cp)

        tvec = plsc.load_gather(t_st, [nbase + iota])

        def k_logits(k, m):
            srcv = plsc.load_gather(src_st, [iota32 + k])
            sg = plsc.load_gather(s_st, [srcv])
            u0g = plsc.load_gather(u0_st, [srcv])
            u1g = plsc.load_gather(u1_st, [srcv])
            e = sg + tvec
            e = jnp.where(e >= 0.0, e, e * jnp.float32(0.01))
            e_st[pl.ds(k * 16, 16)] = e
            u0g_st[pl.ds(k * 16, 16)] = u0g
            u1g_st[pl.ds(k * 16, 16)] = u1g
            return jnp.maximum(m, e)

        m = lax.fori_loop(0, K, k_logits, jnp.full((16,), -1e30, jnp.float32))

        def k_exp(k, acc):
            p = jnp.exp(e_st[pl.ds(k * 16, 16)] - m)
            e_st[pl.ds(k * 16, 16)] = p
            return acc + p

        denom = lax.fori_loop(0, K, k_exp, jnp.zeros((16,), jnp.float32))
        inv = jnp.float32(1.0) / denom

        def k_alpha(k, c):
            a = e_st[pl.ds(k * 16, 16)] * inv
            e_st[pl.ds(k * 16, 16)] = a
            wck = plsc.load_gather(wcol_st, [jnp.full((16,), k, jnp.int32)])
            plsc.store_scatter(w_st, [iota32 + k], a * wck)
            return c

        lax.fori_loop(0, K, k_alpha, 0)

        def k_row(k, c):
            a0 = e_st[pl.ds(k * 16, 16)]
            a1 = e_st[pl.ds(k * 16 + 16, 16)]
            r = a0 * u0g_st[pl.ds(k * 16, 16)] + a1 * u1g_st[pl.ds(k * 16 + 16, 16)]
            plsc.store_scatter(row_st, [iota32 + k], r)
            return c

        lax.fori_loop(0, K - 1, k_row, 0)
        pltpu.sync_copy(row_st, row_hbm.at[pl.ds(ebase, EPG)])

        for cp in copies:
            cp.wait()

        def node_body(nl, c):
            rbase = nl * K

            def k_acc(k, accs):
                wk = plsc.load_gather(
                    w_st, [jnp.full((16,), rbase + k, jnp.int32)])
                return tuple(
                    accs[dc] + wk * rows_v[rbase + k, pl.ds(dc * 16, 16)]
                    for dc in range(8))

            accs = lax.fori_loop(
                0, K, k_acc,
                tuple(jnp.zeros((16,), jnp.float32) for _ in range(8)))
            for dc in range(8):
                col_st[pl.ds(nl * D + dc * 16, 16)] = accs[dc]
            return c

        lax.fori_loop(0, GN, node_body, 0)
        pltpu.sync_copy(col_st, col_hbm.at[pl.ds(nbase * D, GN * D)])
        return carry

    lax.fori_loop(0, ngroups, group_body, 0)


def _sc(src, aux, z, wcol):
    fn = functools.partial(
        pl.kernel,
        out_type=[
            jax.ShapeDtypeStruct((N * K,), jnp.float32),
            jax.ShapeDtypeStruct((N * D,), jnp.float32),
        ],
        mesh=plsc.VectorSubcoreMesh(core_axis_name="c", subcore_axis_name="s"),
        compiler_params=pltpu.CompilerParams(needs_layout_passes=False),
        scratch_types=[
            pltpu.VMEM((N,), jnp.float32),
            pltpu.VMEM((N,), jnp.float32),
            pltpu.VMEM((N,), jnp.float32),
            pltpu.VMEM((N,), jnp.float32),
            pltpu.VMEM((K,), jnp.float32),
            pltpu.VMEM((EPG,), jnp.int32),
            pltpu.VMEM((EPG,), jnp.float32),
            pltpu.VMEM((EPG,), jnp.float32),
            pltpu.VMEM((EPG,), jnp.float32),
            pltpu.VMEM((EPG,), jnp.float32),
            pltpu.VMEM((EPG,), jnp.float32),
            pltpu.VMEM((EPG, D), jnp.float32),
            pltpu.VMEM((GN * D,), jnp.float32),
            pltpu.SemaphoreType.DMA,
        ],
    )(_sc_body)
    return fn(src, aux, z, wcol)


def _bn_body(row_ref, col_ref, gr_ref, br_ref, gc_ref, bc_ref, out_ref):
    r = row_ref[...][:, :K - 1]
    mr = jnp.mean(r)
    vr = jnp.mean((r - mr) * (r - mr))
    rb = (r - mr) * lax.rsqrt(vr + 1e-5) * gr_ref[0, 0] + br_ref[0, 0]
    c = col_ref[...]
    mc = jnp.mean(c)
    vc = jnp.mean((c - mc) * (c - mc))
    cb = (c - mc) * lax.rsqrt(vc + 1e-5) * gc_ref[0, 0] + bc_ref[0, 0]
    out_ref[...] = jnp.concatenate(
        [jnp.maximum(rb, 0.0), jnp.maximum(cb, 0.0)], axis=1)


def _bn(row_raw, col_raw, gr, br, gc, bc):
    return pl.pallas_call(
        _bn_body,
        out_shape=jax.ShapeDtypeStruct((N, K - 1 + D), jnp.float32),
    )(row_raw, col_raw, gr, br, gc, bc)


def kernel(h, edge_index, W, A, Wrow, brow, gamma_row, beta_row,
           Wcol, bcol, gamma_col, beta_col):
    del brow, bcol
    src = edge_index[0]
    wt = W.T
    baux = jnp.concatenate(
        [A.reshape(2, D), Wrow, jnp.zeros((4, D), jnp.float32)], axis=0)
    z, aux3 = _dense(h, wt, baux)
    aux = aux3.transpose(1, 0, 2).reshape(8, N)
    row_flat, col_flat = _sc(src, aux, z, Wcol)
    return _bn(row_flat.reshape(N, K), col_flat.reshape(N, D),
               gamma_row.reshape(1, 1), beta_row.reshape(1, 1),
               gamma_col.reshape(1, 1), beta_col.reshape(1, 1))
